# baseline (speedup 1.0000x reference)
import functools

import jax
import jax.numpy as jnp
from jax import lax
from jax.experimental import pallas as pl
from jax.experimental.pallas import tpu as pltpu
from jax.experimental.pallas import tpu_sc as plsc

_NC, _NS, _L = 2, 16, 16
_NW = _NC * _NS
_C = 256
_R = 4

_TB = 512


def _sc_gather(idx, table, n, d):
    n_chunks = n // (_NW * _C)
    mesh = plsc.VectorSubcoreMesh(core_axis_name="c", subcore_axis_name="s")

    @functools.partial(
        pl.kernel,
        out_type=jax.ShapeDtypeStruct((n, d), jnp.float32),
        mesh=mesh,
        compiler_params=pltpu.CompilerParams(use_tc_tiling_on_sc=False),
        scratch_types=[
            pltpu.VMEM((_R, _C), jnp.int32),
            pltpu.VMEM((_R, _C, 64), jnp.float32),
            pltpu.SemaphoreType.DMA((_R,)),
            pltpu.SemaphoreType.DMA((_R,)),
            pltpu.SemaphoreType.DMA((_R,)),
        ],
    )
    def run(table_hbm, idx_hbm, o_hbm, idx_v, g_v, s_i, s_g, s_o):
        wid = lax.axis_index("s") * _NC + lax.axis_index("c")
        base = wid * (n_chunks * _C)

        def start_idx(i, p):
            off = base + i * _C
            pltpu.async_copy(idx_hbm.at[pl.ds(off, _C)], idx_v.at[p], s_i.at[p])

        def fire_gather(i, p):
            pltpu.make_async_copy(idx_hbm.at[pl.ds(0, _C)], idx_v.at[p],
                                  s_i.at[p]).wait()
            for c in range(0, _C, _L):
                idx_v[p, pl.ds(c, _L)] = idx_v[p, pl.ds(c, _L)] + 1
            pltpu.async_copy(table_hbm.at[idx_v.at[p]], g_v.at[p], s_g.at[p])

        def drain_out(i, p):
            off = base + i * _C
            pltpu.make_async_copy(table_hbm.at[idx_v.at[p]], g_v.at[p],
                                  s_g.at[p]).wait()
            pltpu.async_copy(g_v.at[p], o_hbm.at[pl.ds(off, _C)], s_o.at[p])

        def wait_out(p):
            pltpu.make_async_copy(g_v.at[p], o_hbm.at[pl.ds(0, _C)],
                                  s_o.at[p]).wait()

        start_idx(0, 0)
        start_idx(1, 1)
        fire_gather(0, 0)

        @pl.loop(0, n_chunks, step=_R)
        def _(i0):
            for j in range(_R):
                i = i0 + j
                pa = (j + 2) % _R
                pb = (j + 1) % _R

                @pl.when(i + 2 < n_chunks)
                def _():
                    @pl.when(i + 2 >= _R)
                    def _():
                        wait_out(pa)
                    start_idx(i + 2, pa)

                @pl.when(i + 1 < n_chunks)
                def _():
                    fire_gather(i + 1, pb)

                drain_out(i, j)

        for p in range(_R):
            wait_out(p)

    return run(table, idx)


def _tc_add(g2d, ut, S, D, B):
    H = _TB // 2

    def body(g_ref, u_ref, o_ref):
        g = g_ref[...]
        gt = g.T
        o_ref[0, :, 0:H] = gt[0:D, :] + u_ref[0, :, 0:H]
        o_ref[0, :, H:_TB] = gt[D:2 * D, :] + u_ref[0, :, H:_TB]

    grid = (S, B // _TB)
    return pl.pallas_call(
        body,
        grid=grid,
        in_specs=[
            pl.BlockSpec((_TB * D // 128, 128),
                         lambda l, bb, nb=B // _TB: (l * nb + bb, 0)),
            pl.BlockSpec((1, D, _TB), lambda l, bb: (l, 0, bb)),
        ],
        out_specs=pl.BlockSpec((1, D, _TB), lambda l, bb: (l, 0, bb)),
        out_shape=jax.ShapeDtypeStruct((S, D, B), jnp.float32),
    )(g2d, ut)


def kernel(z, u, table):
    B, S = z.shape
    D = table.shape[1]
    N = B * S

    zt = z.T
    ut = jnp.transpose(u, (1, 2, 0))

    idx = (zt.astype(jnp.int32)
             .reshape(S, B // _TB, 2, _TB // 2)
             .swapaxes(2, 3)
             .reshape(N))
    g = _sc_gather(idx, table, N, D)
    g2d = g.reshape(N * D // 128, 128)

    vt = _tc_add(g2d, ut, S, D, B)
    v = jnp.transpose(vt, (2, 0, 1))
    return (z, v)


# speedup vs baseline: 1.6786x; 1.6786x over previous
import functools

import jax
import jax.numpy as jnp
from jax import lax
from jax.experimental import pallas as pl
from jax.experimental.pallas import tpu as pltpu
from jax.experimental.pallas import tpu_sc as plsc

_NC, _NS, _L = 2, 16, 16
_NW = _NC * _NS
_C = 256
_R = 4

_TB = 512


def _sc_gather(idx, table, n, d):
    n_chunks = n // (_NW * _C)
    mesh = plsc.VectorSubcoreMesh(core_axis_name="c", subcore_axis_name="s")

    @functools.partial(
        pl.kernel,
        out_type=jax.ShapeDtypeStruct((n, d), jnp.float32),
        mesh=mesh,
        compiler_params=pltpu.CompilerParams(use_tc_tiling_on_sc=False),
        scratch_types=[
            pltpu.VMEM((_R, _C), jnp.int32),
            pltpu.VMEM((_R, _C, 64), jnp.float32),
            pltpu.SemaphoreType.DMA((_R,)),
            pltpu.SemaphoreType.DMA((_R,)),
            pltpu.SemaphoreType.DMA((_R,)),
        ],
    )
    def run(table_hbm, idx_hbm, o_hbm, idx_v, g_v, s_i, s_g, s_o):
        wid = lax.axis_index("s") * _NC + lax.axis_index("c")
        base = wid * (n_chunks * _C)

        def start_idx(i, p):
            off = base + i * _C
            pltpu.async_copy(idx_hbm.at[pl.ds(off, _C)], idx_v.at[p], s_i.at[p])

        def fire_gather(i, p):
            pltpu.make_async_copy(idx_hbm.at[pl.ds(0, _C)], idx_v.at[p],
                                  s_i.at[p]).wait()
            for c in range(0, _C, _L):
                idx_v[p, pl.ds(c, _L)] = idx_v[p, pl.ds(c, _L)] + 1
            pltpu.async_copy(table_hbm.at[idx_v.at[p]], g_v.at[p], s_g.at[p])

        def drain_out(i, p):
            off = base + i * _C
            pltpu.make_async_copy(table_hbm.at[idx_v.at[p]], g_v.at[p],
                                  s_g.at[p]).wait()
            pltpu.async_copy(g_v.at[p], o_hbm.at[pl.ds(off, _C)], s_o.at[p])

        def wait_out(p):
            pltpu.make_async_copy(g_v.at[p], o_hbm.at[pl.ds(0, _C)],
                                  s_o.at[p]).wait()

        start_idx(0, 0)
        start_idx(1, 1)
        fire_gather(0, 0)

        @pl.loop(0, n_chunks, step=_R)
        def _(i0):
            for j in range(_R):
                i = i0 + j
                pa = (j + 2) % _R
                pb = (j + 1) % _R

                @pl.when(i + 2 < n_chunks)
                def _():
                    @pl.when(i + 2 >= _R)
                    def _():
                        wait_out(pa)
                    start_idx(i + 2, pa)

                @pl.when(i + 1 < n_chunks)
                def _():
                    fire_gather(i + 1, pb)

                drain_out(i, j)

        for p in range(_R):
            wait_out(p)

    return run(table, idx)


def _tc_add(g2d, ut, S, D, B):
    H = _TB // 2
    BB = 8192
    K = BB // _TB
    GR = _TB * D // 128

    def body(g_ref, u_ref, o_ref):
        for k in range(K):
            gt = g_ref[k * GR:(k + 1) * GR, :].T
            b0 = k * _TB
            o_ref[0, :, b0:b0 + H] = gt[0:D, :] + u_ref[0, :, b0:b0 + H]
            o_ref[0, :, b0 + H:b0 + _TB] = (
                gt[D:2 * D, :] + u_ref[0, :, b0 + H:b0 + _TB])

    grid = (S, B // BB)
    return pl.pallas_call(
        body,
        grid=grid,
        in_specs=[
            pl.BlockSpec((BB * D // 128, 128),
                         lambda l, bb, nb=B // BB: (l * nb + bb, 0)),
            pl.BlockSpec((1, D, BB), lambda l, bb: (l, 0, bb)),
        ],
        out_specs=pl.BlockSpec((1, D, BB), lambda l, bb: (l, 0, bb)),
        out_shape=jax.ShapeDtypeStruct((S, D, B), jnp.float32),
    )(g2d, ut)


def kernel(z, u, table):
    B, S = z.shape
    D = table.shape[1]
    N = B * S

    zt = z.T
    ut = jnp.transpose(u, (1, 2, 0))

    idx = (zt.astype(jnp.int32)
             .reshape(S, B // _TB, 2, _TB // 2)
             .swapaxes(2, 3)
             .reshape(N))
    g = _sc_gather(idx, table, N, D)
    g2d = g.reshape(N * D // 128, 128)

    vt = _tc_add(g2d, ut, S, D, B)
    v = jnp.transpose(vt, (2, 0, 1))
    return (z, v)
